# baseline (speedup 1.0000x reference)
import functools

import jax
import jax.numpy as jnp
from jax.experimental import pallas as pl

_BM = 200


def _support_body(x_ref, w_ref, sup_ref, u_ref, *, uw):
    s = jnp.dot(x_ref[...], w_ref[...], preferred_element_type=jnp.float32)
    sup_ref[...] = s
    u_ref[...] = s[:, :uw].astype(jnp.bfloat16)


def _support(x, w, uw):
    n = x.shape[0]
    f = w.shape[1]
    return pl.pallas_call(
        functools.partial(_support_body, uw=uw),
        out_shape=(
            jax.ShapeDtypeStruct((n, f), jnp.float32),
            jax.ShapeDtypeStruct((n, uw), jnp.bfloat16),
        ),
    )(x, w)


def _fused_body(*refs, side_len, relu, avg, cast, emit, nxt, uw):
    it = iter(refs)
    adj_ref = next(it)
    u_ref = next(it)
    sup_ref = next(it)
    b_ref = next(it)
    wn_ref = next(it) if nxt else None
    res_ref = next(it) if avg else None
    adjout_ref = next(it) if cast else None
    out_ref = next(it) if emit else None
    supn_ref = next(it) if nxt else None
    un_ref = next(it) if nxt else None

    adj = adj_ref[...]
    if cast:
        adj = adj.astype(jnp.bfloat16)
        adjout_ref[...] = adj
    s1 = jnp.dot(adj, u_ref[...], preferred_element_type=jnp.float32)
    sup = sup_ref[...]
    z = jnp.concatenate([s1, sup[:, s1.shape[1]:]], axis=1)
    if side_len != s1.shape[1]:
        col = jax.lax.broadcasted_iota(jnp.int32, z.shape, 1)
        z = jnp.where(col < side_len, z, sup)
    y = z + b_ref[...]
    if relu:
        y = jnp.maximum(y, 0.0)
    if avg:
        y = (res_ref[...] + y) * 0.5
    if emit:
        out_ref[...] = y
    if nxt:
        sn = jnp.dot(y, wn_ref[...], preferred_element_type=jnp.float32)
        supn_ref[...] = sn
        un_ref[...] = sn[:, :uw].astype(jnp.bfloat16)


def _fused_layer(adj_in, u, sup, b, wn, res, bm, side_len, relu, cast, emit):
    n, f = sup.shape
    uw = u.shape[1]
    nxt = wn is not None

    def row(shape, dtype):
        return pl.BlockSpec((bm,) + shape[1:], lambda i: (i,) + (0,) * (len(shape) - 1)), \
            jax.ShapeDtypeStruct(shape, dtype)

    in_specs = [
        pl.BlockSpec((bm, n), lambda i: (i, 0)),
        pl.BlockSpec((n, uw), lambda i: (0, 0)),
        pl.BlockSpec((bm, f), lambda i: (i, 0)),
        pl.BlockSpec((1, f), lambda i: (0, 0)),
    ]
    args = [adj_in, u, sup, jnp.reshape(b, (1, f))]
    if nxt:
        in_specs.append(pl.BlockSpec(wn.shape, lambda i: (0, 0)))
        args.append(wn)
    if res is not None:
        in_specs.append(pl.BlockSpec((bm, f), lambda i: (i, 0)))
        args.append(res)

    out_specs = []
    out_shape = []
    if cast:
        s, sh = row((n, n), jnp.bfloat16)
        out_specs.append(s)
        out_shape.append(sh)
    if emit:
        s, sh = row((n, f), jnp.float32)
        out_specs.append(s)
        out_shape.append(sh)
    if nxt:
        fn = wn.shape[1]
        s, sh = row((n, fn), jnp.float32)
        out_specs.append(s)
        out_shape.append(sh)
        s, sh = row((n, uw), jnp.bfloat16)
        out_specs.append(s)
        out_shape.append(sh)

    outs = pl.pallas_call(
        functools.partial(
            _fused_body, side_len=side_len, relu=relu, avg=res is not None,
            cast=cast, emit=emit, nxt=nxt, uw=uw,
        ),
        grid=(n // bm,),
        in_specs=in_specs,
        out_specs=tuple(out_specs),
        out_shape=tuple(out_shape),
    )(*args)
    return list(outs)


def kernel(features, adj, Ws, bs):
    n = features.shape[0]
    h = Ws[0].shape[1]
    out_d = Ws[-1].shape[1]
    sl = max(h // 3, 2)
    bm = _BM if n % _BM == 0 else n

    w_last = jnp.pad(Ws[13], ((0, 0), (0, 128 - out_d)))
    b_last = jnp.pad(bs[13], ((0, 128 - out_d),))

    sup, u = _support(features, Ws[0], sl)
    adj_bf, sup, u = _fused_layer(
        adj, u, sup, bs[0], Ws[1], None, bm, sl, True, True, False)
    feats, sup, u = _fused_layer(
        adj_bf, u, sup, bs[1], Ws[2], features[:, :h], bm, sl, True, False, True)
    for i in (2, 4, 6, 8, 10):
        sup, u = _fused_layer(
            adj_bf, u, sup, bs[i], Ws[i + 1], None, bm, sl, True, False, False)
        feats, sup, u = _fused_layer(
            adj_bf, u, sup, bs[i + 1], Ws[i + 2], feats, bm, sl, True, False, True)
    feats, sup, u = _fused_layer(
        adj_bf, u, sup, bs[12], w_last, feats, bm, sl, True, False, True)
    (coords_p,) = _fused_layer(
        adj_bf, u, sup, b_last, None, None, bm, max(out_d // 3, 2), False,
        False, True)
    return coords_p[:, :out_d], feats


# speedup vs baseline: 1.2657x; 1.2657x over previous
import functools

import jax
import jax.numpy as jnp
from jax.experimental import pallas as pl
from jax.experimental.pallas import tpu as pltpu

_BM = 400


def _support_body(x_ref, w_ref, sup_ref, u_ref, *, uw):
    s = jnp.dot(x_ref[...], w_ref[...], preferred_element_type=jnp.float32)
    sup_ref[...] = s
    u_ref[...] = s[:, :uw].astype(jnp.bfloat16)


def _support(x, w, uw):
    n = x.shape[0]
    f = w.shape[1]
    return pl.pallas_call(
        functools.partial(_support_body, uw=uw),
        out_shape=(
            jax.ShapeDtypeStruct((n, f), jnp.float32),
            jax.ShapeDtypeStruct((n, uw), jnp.bfloat16),
        ),
    )(x, w)


def _l0_body(adj_ref, u_ref, sup_ref, b_ref, wn_ref, adjout_ref, supn_ref,
             un_ref, *, uw):
    adj = adj_ref[...].astype(jnp.bfloat16)
    adjout_ref[...] = adj
    s1 = jnp.dot(adj, u_ref[...], preferred_element_type=jnp.float32)
    sup = sup_ref[...]
    y = jnp.concatenate([s1, sup[:, uw:]], axis=1) + b_ref[...]
    y = jnp.maximum(y, 0.0)
    sn = jnp.dot(y, wn_ref[...], preferred_element_type=jnp.float32)
    supn_ref[...] = sn
    un_ref[...] = sn[:, :uw].astype(jnp.bfloat16)


def _layer0(adj, u, sup, b, wn, bm, uw):
    n, f = sup.shape
    return pl.pallas_call(
        functools.partial(_l0_body, uw=uw),
        grid=(n // bm,),
        in_specs=[
            pl.BlockSpec((bm, n), lambda i: (i, 0)),
            pl.BlockSpec((n, uw), lambda i: (0, 0)),
            pl.BlockSpec((bm, f), lambda i: (i, 0)),
            pl.BlockSpec((1, f), lambda i: (0, 0)),
            pl.BlockSpec(wn.shape, lambda i: (0, 0)),
        ],
        out_specs=(
            pl.BlockSpec((bm, n), lambda i: (i, 0)),
            pl.BlockSpec((bm, f), lambda i: (i, 0)),
            pl.BlockSpec((bm, uw), lambda i: (i, 0)),
        ),
        out_shape=(
            jax.ShapeDtypeStruct((n, n), jnp.bfloat16),
            jax.ShapeDtypeStruct((n, f), jnp.float32),
            jax.ShapeDtypeStruct((n, uw), jnp.bfloat16),
        ),
    )(adj, u, sup, jnp.reshape(b, (1, f)), wn)


def _mega_body(adj_ref, u1_ref, sup_in_ref, res_in_ref, w_ref, b_ref,
               feats_out_ref, coords_ref, sup_s, feats_s, u_s, ustg_s,
               *, bm, sl, nblk, nlay):
    l = pl.program_id(0)
    i = pl.program_id(1)
    rows = pl.ds(i * bm, bm)
    last_l = nlay - 1

    @pl.when(jnp.logical_and(l == 0, i == 0))
    def _():
        u_s[...] = u1_ref[...]

    @pl.when(l == 0)
    def _():
        sup_s[rows, :] = sup_in_ref[...]
        feats_s[rows, :] = res_in_ref[...]

    s1 = jnp.dot(adj_ref[...], u_s[...], preferred_element_type=jnp.float32)
    sup = sup_s[rows, :]
    z = jnp.concatenate([s1, sup[:, sl:]], axis=1)
    col = jax.lax.broadcasted_iota(jnp.int32, z.shape, 1)
    thresh = jnp.where(l == last_l, 2, sl)
    z = jnp.where(col < thresh, z, sup)
    y = z + b_ref[...]
    y = jnp.where(l == last_l, y, jnp.maximum(y, 0.0))
    avg_f = jnp.logical_or(
        jnp.logical_and(l % 2 == 0, l <= last_l - 2), l == last_l - 1)
    y = jnp.where(avg_f, (feats_s[rows, :] + y) * 0.5, y)

    @pl.when(avg_f)
    def _():
        feats_s[rows, :] = y
    feats_out_ref[...] = feats_s[rows, :]

    @pl.when(l == last_l)
    def _():
        coords_ref[...] = y

    @pl.when(l < last_l)
    def _():
        sn = jnp.dot(y, w_ref[...], preferred_element_type=jnp.float32)
        sup_s[rows, :] = sn
        ustg_s[rows, :] = sn[:, :sl].astype(jnp.bfloat16)

    @pl.when(jnp.logical_and(i == nblk - 1, l < last_l))
    def _():
        u_s[...] = ustg_s[...]


def _megalayers(adj_bf, u1, sup1, res0, wstk, bstk, bm, sl):
    n, f = sup1.shape
    nblk = n // bm
    nlay = wstk.shape[0]
    return pl.pallas_call(
        functools.partial(_mega_body, bm=bm, sl=sl, nblk=nblk, nlay=nlay),
        grid=(nlay, nblk),
        in_specs=[
            pl.BlockSpec((bm, n), lambda l, i: (i, 0)),
            pl.BlockSpec((n, sl), lambda l, i: (0, 0)),
            pl.BlockSpec((bm, f), lambda l, i: (jnp.where(l == 0, i, 0), 0)),
            pl.BlockSpec((bm, f), lambda l, i: (jnp.where(l == 0, i, 0), 0)),
            pl.BlockSpec((None, f, f), lambda l, i: (l, 0, 0)),
            pl.BlockSpec((None, 1, f), lambda l, i: (l, 0, 0)),
        ],
        out_specs=(
            pl.BlockSpec(
                (bm, f), lambda l, i: (jnp.where(l >= nlay - 2, i, 0), 0)),
            pl.BlockSpec(
                (bm, f), lambda l, i: (jnp.where(l == nlay - 1, i, 0), 0)),
        ),
        out_shape=(
            jax.ShapeDtypeStruct((n, f), jnp.float32),
            jax.ShapeDtypeStruct((n, f), jnp.float32),
        ),
        scratch_shapes=[
            pltpu.VMEM((n, f), jnp.float32),
            pltpu.VMEM((n, f), jnp.float32),
            pltpu.VMEM((n, sl), jnp.bfloat16),
            pltpu.VMEM((n, sl), jnp.bfloat16),
        ],
    )(adj_bf, u1, sup1, res0, wstk, bstk)


def kernel(features, adj, Ws, bs):
    n = features.shape[0]
    h = Ws[0].shape[1]
    out_d = Ws[-1].shape[1]
    sl = max(h // 3, 2)
    bm = _BM if n % _BM == 0 else n

    w_last = jnp.pad(Ws[13], ((0, 0), (0, h - out_d)))
    b_last = jnp.pad(bs[13], ((0, h - out_d),))
    wstk = jnp.stack([Ws[i] for i in range(2, 13)] + [w_last, jnp.zeros_like(w_last)])
    bstk = jnp.stack(
        [jnp.reshape(b, (1, h)) for b in bs[1:13]]
        + [jnp.reshape(b_last, (1, h))])

    sup, u = _support(features, Ws[0], sl)
    adj_bf, sup, u = _layer0(adj, u, sup, bs[0], Ws[1], bm, sl)
    feats, coords_p = _megalayers(
        adj_bf, u, sup, features[:, :h], wstk, bstk, bm, sl)
    return coords_p[:, :out_d], feats
